# initial kernel scaffold (unmeasured)
import jax
import jax.numpy as jnp
from jax import lax
from jax.experimental import pallas as pl
from jax.experimental.pallas import tpu as pltpu

Y = 4
M = 4096
D = 4096


def _allgather_y(x):

    def body(x_ref, gathered_ref, local_sem, send_sems, recv_sems):
        my_x = lax.axis_index("x")
        my_y = lax.axis_index("y")
        my_z = lax.axis_index("z")

        barrier_sem = pltpu.get_barrier_semaphore()
        for dy in range(1, Y):
            peer = (my_y + dy) % Y
            pl.semaphore_signal(
                barrier_sem,
                inc=1,
                device_id=(my_x, peer, my_z),
                device_id_type=pl.DeviceIdType.MESH,
            )
        pl.semaphore_wait(barrier_sem, Y - 1)

        local_cp = pltpu.make_async_copy(x_ref, gathered_ref.at[my_y], local_sem)
        local_cp.start()

        sends = []
        for dy in range(1, Y):
            peer = (my_y + dy) % Y
            rdma = pltpu.make_async_remote_copy(
                src_ref=x_ref,
                dst_ref=gathered_ref.at[my_y],
                send_sem=send_sems.at[dy - 1],
                recv_sem=recv_sems.at[dy - 1],
                device_id=(my_x, peer, my_z),
                device_id_type=pl.DeviceIdType.MESH,
            )
            rdma.start()
            sends.append(rdma)

        for dy in range(1, Y):
            src = (my_y - dy) % Y
            recv = pltpu.make_async_remote_copy(
                src_ref=x_ref,
                dst_ref=gathered_ref.at[src],
                send_sem=send_sems.at[dy - 1],
                recv_sem=recv_sems.at[dy - 1],
                device_id=(my_x, my_y, my_z),
                device_id_type=pl.DeviceIdType.MESH,
            )
            recv.wait_recv()

        for rdma in sends:
            rdma.wait_send()
        local_cp.wait()

    return pl.pallas_call(
        body,
        out_shape=jax.ShapeDtypeStruct((Y, M, D), jnp.float32),
        in_specs=[pl.BlockSpec(memory_space=pltpu.ANY)],
        out_specs=pl.BlockSpec(memory_space=pltpu.ANY),
        scratch_shapes=[
            pltpu.SemaphoreType.DMA,
            pltpu.SemaphoreType.DMA((Y - 1,)),
            pltpu.SemaphoreType.DMA((Y - 1,)),
        ],
        compiler_params=pltpu.CompilerParams(collective_id=0),
    )(x)


def _compute(gathered, resid, gamma2d):
    BLK = 256

    def body(g_ref, r_ref, gm_ref, o_ref):
        g = g_ref[...]
        y = (g[0] + g[1]) + (g[2] + g[3]) + r_ref[...]
        ms = jnp.mean(y * y, axis=-1, keepdims=True)
        o_ref[...] = y * lax.rsqrt(ms + 1e-6) * gm_ref[...]

    return pl.pallas_call(
        body,
        grid=(M // BLK,),
        in_specs=[
            pl.BlockSpec((Y, BLK, D), lambda i: (0, i, 0)),
            pl.BlockSpec((BLK, D), lambda i: (i, 0)),
            pl.BlockSpec((1, D), lambda i: (0, 0)),
        ],
        out_specs=pl.BlockSpec((BLK, D), lambda i: (i, 0)),
        out_shape=jax.ShapeDtypeStruct((M, D), jnp.float32),
    )(gathered, resid, gamma2d)


def kernel(partial, resid, gamma):
    x = partial.reshape(M, D)
    gathered = _allgather_y(x)
    return _compute(gathered, resid, gamma.reshape(1, D))


# baseline (device time: 3003685 ns/iter reference)
import jax
import jax.numpy as jnp
from jax import lax
from jax.experimental import pallas as pl
from jax.experimental.pallas import tpu as pltpu

Y = 4
M = 4096
D = 4096


def _allgather_y(x):

    def body(x_ref, gathered_ref, local_sem, send_sems, recv_sems):
        my_x = lax.axis_index("x")
        my_y = lax.axis_index("y")
        my_z = lax.axis_index("z")

        barrier_sem = pltpu.get_barrier_semaphore()
        for dy in range(1, Y):
            peer = (my_y + dy) % Y
            pl.semaphore_signal(
                barrier_sem,
                inc=1,
                device_id=(my_x, peer, my_z),
                device_id_type=pl.DeviceIdType.MESH,
            )
        pl.semaphore_wait(barrier_sem, Y - 1)

        local_cp = pltpu.make_async_copy(x_ref, gathered_ref.at[my_y], local_sem)
        local_cp.start()

        sends = []
        for dy in range(1, Y):
            peer = (my_y + dy) % Y
            rdma = pltpu.make_async_remote_copy(
                src_ref=x_ref,
                dst_ref=gathered_ref.at[my_y],
                send_sem=send_sems.at[dy - 1],
                recv_sem=recv_sems.at[dy - 1],
                device_id=(my_x, peer, my_z),
                device_id_type=pl.DeviceIdType.MESH,
            )
            rdma.start()
            sends.append(rdma)

        for dy in range(1, Y):
            src = (my_y - dy) % Y
            recv = pltpu.make_async_remote_copy(
                src_ref=x_ref,
                dst_ref=gathered_ref.at[src],
                send_sem=send_sems.at[dy - 1],
                recv_sem=recv_sems.at[dy - 1],
                device_id=(my_x, my_y, my_z),
                device_id_type=pl.DeviceIdType.MESH,
            )
            recv.wait_recv()

        for rdma in sends:
            rdma.wait_send()
        local_cp.wait()

    return pl.pallas_call(
        body,
        out_shape=jax.ShapeDtypeStruct((Y, M, D), jnp.float32),
        in_specs=[pl.BlockSpec(memory_space=pl.ANY)],
        out_specs=pl.BlockSpec(memory_space=pl.ANY),
        scratch_shapes=[
            pltpu.SemaphoreType.DMA,
            pltpu.SemaphoreType.DMA((Y - 1,)),
            pltpu.SemaphoreType.DMA((Y - 1,)),
        ],
        compiler_params=pltpu.CompilerParams(collective_id=0),
    )(x)


def _compute(gathered, resid, gamma2d):
    BLK = 128

    def body(g_ref, r_ref, gm_ref, o_ref):
        g = g_ref[...]
        y = (g[0] + g[1]) + (g[2] + g[3]) + r_ref[...]
        ms = jnp.mean(y * y, axis=-1, keepdims=True)
        o_ref[...] = y * lax.rsqrt(ms + 1e-6) * gm_ref[...]

    return pl.pallas_call(
        body,
        grid=(M // BLK,),
        in_specs=[
            pl.BlockSpec((Y, BLK, D), lambda i: (0, i, 0)),
            pl.BlockSpec((BLK, D), lambda i: (i, 0)),
            pl.BlockSpec((1, D), lambda i: (0, 0)),
        ],
        out_specs=pl.BlockSpec((BLK, D), lambda i: (i, 0)),
        out_shape=jax.ShapeDtypeStruct((M, D), jnp.float32),
    )(gathered, resid, gamma2d)


def kernel(partial, resid, gamma):
    x = partial.reshape(M, D)
    gathered = _allgather_y(x)
    return _compute(gathered, resid, gamma.reshape(1, D))


# device time: 449192 ns/iter; 6.6869x vs baseline; 6.6869x over previous
import jax
import jax.numpy as jnp
from jax import lax
from jax.experimental import pallas as pl
from jax.experimental.pallas import tpu as pltpu

Y = 4
M = 4096
D = 4096
NG = 8
R = M // NG


def _pos_to_xz(p):
    x = p // 4
    z = jnp.where(x == 0, p, 7 - p)
    return x, z


def _y_reduce_norm(x_rows, resid_rows, gamma2d):

    def body(x_ref, r_ref, gm_ref, o_ref, recv_buf, send_sems, recv_sems):
        my_x = lax.axis_index("x")
        my_y = lax.axis_index("y")
        my_z = lax.axis_index("z")

        barrier_sem = pltpu.get_barrier_semaphore()
        for dy in range(1, Y):
            peer = (my_y + dy) % Y
            pl.semaphore_signal(
                barrier_sem, inc=1,
                device_id=(my_x, peer, my_z),
                device_id_type=pl.DeviceIdType.MESH,
            )
        pl.semaphore_wait(barrier_sem, Y - 1)

        sends = []
        for dy in range(1, Y):
            peer = (my_y + dy) % Y
            rdma = pltpu.make_async_remote_copy(
                src_ref=x_ref,
                dst_ref=recv_buf.at[dy - 1],
                send_sem=send_sems.at[dy - 1],
                recv_sem=recv_sems.at[dy - 1],
                device_id=(my_x, peer, my_z),
                device_id_type=pl.DeviceIdType.MESH,
            )
            rdma.start()
            sends.append(rdma)
        for dy in range(1, Y):
            recv = pltpu.make_async_remote_copy(
                src_ref=x_ref,
                dst_ref=recv_buf.at[dy - 1],
                send_sem=send_sems.at[dy - 1],
                recv_sem=recv_sems.at[dy - 1],
                device_id=(my_x, my_y, my_z),
                device_id_type=pl.DeviceIdType.MESH,
            )
            recv.wait_recv()

        y = (
            x_ref[...].astype(jnp.float32)
            + recv_buf[0].astype(jnp.float32)
            + recv_buf[1].astype(jnp.float32)
            + recv_buf[2].astype(jnp.float32)
            + r_ref[...]
        )
        ms = jnp.mean(y * y, axis=-1, keepdims=True)
        o_ref[...] = (y * lax.rsqrt(ms + 1e-6) * gm_ref[...]).astype(jnp.bfloat16)

        for rdma in sends:
            rdma.wait_send()

    return pl.pallas_call(
        body,
        out_shape=jax.ShapeDtypeStruct((R, D), jnp.bfloat16),
        in_specs=[
            pl.BlockSpec(memory_space=pltpu.MemorySpace.VMEM),
            pl.BlockSpec(memory_space=pltpu.MemorySpace.VMEM),
            pl.BlockSpec(memory_space=pltpu.MemorySpace.VMEM),
        ],
        out_specs=pl.BlockSpec(memory_space=pltpu.MemorySpace.VMEM),
        scratch_shapes=[
            pltpu.VMEM((Y - 1, R, D), jnp.bfloat16),
            pltpu.SemaphoreType.DMA((Y - 1,)),
            pltpu.SemaphoreType.DMA((Y - 1,)),
        ],
        compiler_params=pltpu.CompilerParams(
            collective_id=0, vmem_limit_bytes=100 * 1024 * 1024
        ),
    )(x_rows, resid_rows, gamma2d)


def _xz_allgather(o_rows):
    N_CW = NG // 2
    N_CCW = NG - 1 - N_CW

    def body(o_ref, out_ref, buf, loc_sem, conv_sems,
             cw_ssem, cw_rsem, ccw_ssem, ccw_rsem, conv):
        my_x = lax.axis_index("x")
        my_y = lax.axis_index("y")
        my_z = lax.axis_index("z")
        p = jnp.where(my_x == 0, my_z, 7 - my_z)
        rx, rz = _pos_to_xz((p + 1) % NG)
        lx, lz = _pos_to_xz((p - 1) % NG)

        barrier_sem = pltpu.get_barrier_semaphore()
        for nx, nz in ((rx, rz), (lx, lz)):
            pl.semaphore_signal(
                barrier_sem, inc=1,
                device_id=(nx, my_y, nz),
                device_id_type=pl.DeviceIdType.MESH,
            )
        pl.semaphore_wait(barrier_sem, 2)

        loc = pltpu.make_async_copy(o_ref, buf.at[p], loc_sem)
        loc.start()
        loc.wait()

        def convert_store(g, slot):
            cp = pltpu.make_async_copy(
                conv.at[slot], out_ref.at[pl.ds(g * R, R)], conv_sems.at[slot]
            )
            conv[slot] = buf[g].astype(jnp.float32)
            cp.start()
            return cp

        def mk(src_blk, dst_blk, ssem, rsem, dev):
            return pltpu.make_async_remote_copy(
                src_ref=buf.at[src_blk],
                dst_ref=buf.at[dst_blk],
                send_sem=ssem,
                recv_sem=rsem,
                device_id=dev,
                device_id_type=pl.DeviceIdType.MESH,
            )

        pending = []
        sends = []
        conv_i = 0
        for h in range(N_CW):
            cw_blk = (p - h) % NG
            cw = mk(cw_blk, cw_blk, cw_ssem.at[h], cw_rsem.at[h], (rx, my_y, rz))
            cw.start()
            sends.append(cw)
            if h < N_CCW:
                ccw_blk = (p + h) % NG
                ccw = mk(ccw_blk, ccw_blk, ccw_ssem.at[h], ccw_rsem.at[h],
                         (lx, my_y, lz))
                ccw.start()
                sends.append(ccw)

            if h == 0:
                todo = [p]
            else:
                todo = [(p - h) % NG]
                if h - 1 < N_CCW:
                    todo.append((p + h) % NG)
            for g in todo:
                if len(pending) == 2:
                    pending.pop(0).wait()
                pending.append(convert_store(g, conv_i % 2))
                conv_i += 1

            mk(cw_blk, (p - 1 - h) % NG, cw_ssem.at[h], cw_rsem.at[h],
               (my_x, my_y, my_z)).wait_recv()
            if h < N_CCW:
                mk(cw_blk, (p + 1 + h) % NG, ccw_ssem.at[h], ccw_rsem.at[h],
                   (my_x, my_y, my_z)).wait_recv()

        for g in ((p - N_CW) % NG, (p + N_CCW) % NG):
            if len(pending) == 2:
                pending.pop(0).wait()
            pending.append(convert_store(g, conv_i % 2))
            conv_i += 1
        for cp in pending:
            cp.wait()
        for s in sends:
            s.wait_send()

    return pl.pallas_call(
        body,
        out_shape=jax.ShapeDtypeStruct((M, D), jnp.float32),
        in_specs=[pl.BlockSpec(memory_space=pltpu.MemorySpace.VMEM)],
        out_specs=pl.BlockSpec(memory_space=pl.ANY),
        scratch_shapes=[
            pltpu.VMEM((NG, R, D), jnp.bfloat16),
            pltpu.SemaphoreType.DMA,
            pltpu.SemaphoreType.DMA((2,)),
            pltpu.SemaphoreType.DMA((N_CW,)),
            pltpu.SemaphoreType.DMA((N_CW,)),
            pltpu.SemaphoreType.DMA((N_CCW,)),
            pltpu.SemaphoreType.DMA((N_CCW,)),
            pltpu.VMEM((2, R, D), jnp.float32),
        ],
        compiler_params=pltpu.CompilerParams(
            collective_id=1, vmem_limit_bytes=100 * 1024 * 1024
        ),
    )(o_rows)


def kernel(partial, resid, gamma):
    my_x = lax.axis_index("x")
    my_z = lax.axis_index("z")
    g = jnp.where(my_x == 0, my_z, 7 - my_z)
    row0 = g * R
    x_rows = lax.dynamic_slice_in_dim(
        partial.reshape(M, D), row0, R, axis=0
    ).astype(jnp.bfloat16)
    resid_rows = lax.dynamic_slice_in_dim(resid, row0, R, axis=0)
    o_rows = _y_reduce_norm(x_rows, resid_rows, gamma.reshape(1, D))
    return _xz_allgather(o_rows)


# device time: 345330 ns/iter; 8.6980x vs baseline; 1.3008x over previous
import jax
import jax.numpy as jnp
from jax import lax
from jax.experimental import pallas as pl
from jax.experimental.pallas import tpu as pltpu

Y = 4
M = 4096
D = 4096
NG = 8
R = M // NG


def _pos_to_xz(p):
    x = p // 4
    z = jnp.where(x == 0, p, 7 - p)
    return x, z


CH = R // Y


def _y_reduce_norm(x_rows, resid_rows, gamma2d):

    def body(x_ref, r_ref, gm_ref, o_ref, acc, rs_buf,
             rs_ssem, rs_rsem, ag_ssem, ag_rsem):
        my_x = lax.axis_index("x")
        my_y = lax.axis_index("y")
        my_z = lax.axis_index("z")
        right = (my_y + 1) % Y
        left = (my_y - 1) % Y

        barrier_sem = pltpu.get_barrier_semaphore()
        for peer in (left, right):
            pl.semaphore_signal(
                barrier_sem, inc=1,
                device_id=(my_x, peer, my_z),
                device_id_type=pl.DeviceIdType.MESH,
            )
        pl.semaphore_wait(barrier_sem, 2)

        acc[...] = x_ref[...]

        def mk(src, dst, ssem, rsem, dev_y):
            return pltpu.make_async_remote_copy(
                src_ref=src, dst_ref=dst, send_sem=ssem, recv_sem=rsem,
                device_id=(my_x, dev_y, my_z),
                device_id_type=pl.DeviceIdType.MESH,
            )

        sends = []
        for s in range(Y - 1):
            c_send = (my_y - s) % Y
            c_recv = (my_y - s - 1) % Y
            rdma = mk(acc.at[pl.ds(c_send * CH, CH)], rs_buf.at[s],
                      rs_ssem.at[s], rs_rsem.at[s], right)
            rdma.start()
            sends.append(rdma)
            mk(rs_buf.at[s], rs_buf.at[s], rs_ssem.at[s], rs_rsem.at[s],
               my_y).wait_recv()
            sl = pl.ds(c_recv * CH, CH)
            acc[sl] = acc[sl] + rs_buf[s]

        c_own = (my_y + 1) % Y
        sl = pl.ds(c_own * CH, CH)
        yv = acc[sl].astype(jnp.float32) + r_ref[sl]
        ms = jnp.mean(yv * yv, axis=-1, keepdims=True)
        o_ref[sl] = (yv * lax.rsqrt(ms + 1e-6) * gm_ref[...]).astype(jnp.bfloat16)

        for s in range(Y - 1):
            c_send = (my_y + 1 - s) % Y
            c_recv = (my_y - s) % Y
            src = o_ref.at[pl.ds(c_send * CH, CH)]
            rdma = mk(src, src, ag_ssem.at[s], ag_rsem.at[s], right)
            rdma.start()
            sends.append(rdma)
            dst = o_ref.at[pl.ds(c_recv * CH, CH)]
            mk(dst, dst, ag_ssem.at[s], ag_rsem.at[s], my_y).wait_recv()

        for rdma in sends:
            rdma.wait_send()

    return pl.pallas_call(
        body,
        out_shape=jax.ShapeDtypeStruct((R, D), jnp.bfloat16),
        in_specs=[
            pl.BlockSpec(memory_space=pltpu.MemorySpace.VMEM),
            pl.BlockSpec(memory_space=pltpu.MemorySpace.VMEM),
            pl.BlockSpec(memory_space=pltpu.MemorySpace.VMEM),
        ],
        out_specs=pl.BlockSpec(memory_space=pltpu.MemorySpace.VMEM),
        scratch_shapes=[
            pltpu.VMEM((R, D), jnp.bfloat16),
            pltpu.VMEM((Y - 1, CH, D), jnp.bfloat16),
            pltpu.SemaphoreType.DMA((Y - 1,)),
            pltpu.SemaphoreType.DMA((Y - 1,)),
            pltpu.SemaphoreType.DMA((Y - 1,)),
            pltpu.SemaphoreType.DMA((Y - 1,)),
        ],
        compiler_params=pltpu.CompilerParams(
            collective_id=0, vmem_limit_bytes=100 * 1024 * 1024
        ),
    )(x_rows, resid_rows, gamma2d)


def _xz_allgather(o_rows):
    N_CW = NG // 2
    N_CCW = NG - 1 - N_CW

    def body(o_ref, out_ref, buf, loc_sem, conv_sems,
             cw_ssem, cw_rsem, ccw_ssem, ccw_rsem, conv):
        my_x = lax.axis_index("x")
        my_y = lax.axis_index("y")
        my_z = lax.axis_index("z")
        p = jnp.where(my_x == 0, my_z, 7 - my_z)
        rx, rz = _pos_to_xz((p + 1) % NG)
        lx, lz = _pos_to_xz((p - 1) % NG)

        barrier_sem = pltpu.get_barrier_semaphore()
        for nx, nz in ((rx, rz), (lx, lz)):
            pl.semaphore_signal(
                barrier_sem, inc=1,
                device_id=(nx, my_y, nz),
                device_id_type=pl.DeviceIdType.MESH,
            )
        pl.semaphore_wait(barrier_sem, 2)

        loc = pltpu.make_async_copy(o_ref, buf.at[p], loc_sem)
        loc.start()
        loc.wait()

        def convert_store(g, slot):
            cp = pltpu.make_async_copy(
                conv.at[slot], out_ref.at[pl.ds(g * R, R)], conv_sems.at[slot]
            )
            conv[slot] = buf[g].astype(jnp.float32)
            cp.start()
            return cp

        def mk(src_blk, dst_blk, ssem, rsem, dev):
            return pltpu.make_async_remote_copy(
                src_ref=buf.at[src_blk],
                dst_ref=buf.at[dst_blk],
                send_sem=ssem,
                recv_sem=rsem,
                device_id=dev,
                device_id_type=pl.DeviceIdType.MESH,
            )

        pending = []
        sends = []
        conv_i = 0
        for h in range(N_CW):
            cw_blk = (p - h) % NG
            cw = mk(cw_blk, cw_blk, cw_ssem.at[h], cw_rsem.at[h], (rx, my_y, rz))
            cw.start()
            sends.append(cw)
            if h < N_CCW:
                ccw_blk = (p + h) % NG
                ccw = mk(ccw_blk, ccw_blk, ccw_ssem.at[h], ccw_rsem.at[h],
                         (lx, my_y, lz))
                ccw.start()
                sends.append(ccw)

            if h == 0:
                todo = [p]
            else:
                todo = [(p - h) % NG]
                if h - 1 < N_CCW:
                    todo.append((p + h) % NG)
            for g in todo:
                if len(pending) == 2:
                    pending.pop(0).wait()
                pending.append(convert_store(g, conv_i % 2))
                conv_i += 1

            mk(cw_blk, (p - 1 - h) % NG, cw_ssem.at[h], cw_rsem.at[h],
               (my_x, my_y, my_z)).wait_recv()
            if h < N_CCW:
                mk(cw_blk, (p + 1 + h) % NG, ccw_ssem.at[h], ccw_rsem.at[h],
                   (my_x, my_y, my_z)).wait_recv()

        for g in ((p - N_CW) % NG, (p + N_CCW) % NG):
            if len(pending) == 2:
                pending.pop(0).wait()
            pending.append(convert_store(g, conv_i % 2))
            conv_i += 1
        for cp in pending:
            cp.wait()
        for s in sends:
            s.wait_send()

    return pl.pallas_call(
        body,
        out_shape=jax.ShapeDtypeStruct((M, D), jnp.float32),
        in_specs=[pl.BlockSpec(memory_space=pltpu.MemorySpace.VMEM)],
        out_specs=pl.BlockSpec(memory_space=pl.ANY),
        scratch_shapes=[
            pltpu.VMEM((NG, R, D), jnp.bfloat16),
            pltpu.SemaphoreType.DMA,
            pltpu.SemaphoreType.DMA((2,)),
            pltpu.SemaphoreType.DMA((N_CW,)),
            pltpu.SemaphoreType.DMA((N_CW,)),
            pltpu.SemaphoreType.DMA((N_CCW,)),
            pltpu.SemaphoreType.DMA((N_CCW,)),
            pltpu.VMEM((2, R, D), jnp.float32),
        ],
        compiler_params=pltpu.CompilerParams(
            collective_id=1, vmem_limit_bytes=100 * 1024 * 1024
        ),
    )(o_rows)


def kernel(partial, resid, gamma):
    my_x = lax.axis_index("x")
    my_z = lax.axis_index("z")
    g = jnp.where(my_x == 0, my_z, 7 - my_z)
    row0 = g * R
    x_rows = lax.dynamic_slice_in_dim(
        partial.reshape(M, D), row0, R, axis=0
    ).astype(jnp.bfloat16)
    resid_rows = lax.dynamic_slice_in_dim(resid, row0, R, axis=0)
    o_rows = _y_reduce_norm(x_rows, resid_rows, gamma.reshape(1, D))
    return _xz_allgather(o_rows)


# device time: 296568 ns/iter; 10.1281x vs baseline; 1.1644x over previous
import functools

import jax
import jax.numpy as jnp
from jax import lax
from jax.experimental import pallas as pl
from jax.experimental.pallas import tpu as pltpu

Y = 4
M = 4096
D = 4096
NG = 8
R = M // NG
CH = R // Y
N_CW = NG // 2
N_CCW = NG - 1 - N_CW


def _pos_to_xz(p):
    x = p // 4
    z = jnp.where(x == 0, p, 7 - p)
    return x, z


def _fused(x_rows, resid_rows, gamma2d):
    def body(x_ref, r_ref, gm_ref, out_ref, acc, rs_buf, o_chunk, buf, conv,
             rs_ssem, rs_rsem, ag_ssem, ag_rsem, cw_ssem, cw_rsem,
             ccw_ssem, ccw_rsem, loc_sem, conv_sems):
        my_x = lax.axis_index("x")
        my_y = lax.axis_index("y")
        my_z = lax.axis_index("z")
        p = jnp.where(my_x == 0, my_z, 7 - my_z)
        rx, rz = _pos_to_xz((p + 1) % NG)
        lx, lz = _pos_to_xz((p - 1) % NG)
        right_y = (my_y + 1) % Y
        left_y = (my_y - 1) % Y

        neighbors = (
            (my_x, left_y, my_z),
            (my_x, right_y, my_z),
            (rx, my_y, rz),
            (lx, my_y, lz),
        )
        barrier_sem = pltpu.get_barrier_semaphore()
        for dev in neighbors:
            pl.semaphore_signal(
                barrier_sem, inc=1,
                device_id=dev, device_id_type=pl.DeviceIdType.MESH,
            )
        pl.semaphore_wait(barrier_sem, len(neighbors))

        def remote(src, dst, ssem, rsem, dev):
            return pltpu.make_async_remote_copy(
                src_ref=src, dst_ref=dst, send_sem=ssem, recv_sem=rsem,
                device_id=dev, device_id_type=pl.DeviceIdType.MESH,
            )

        sends = []

        acc[...] = x_ref[...]
        for s in range(Y - 1):
            c_send = (my_y - s) % Y
            c_recv = (my_y - s - 1) % Y
            rdma = remote(acc.at[pl.ds(c_send * CH, CH)], rs_buf.at[s],
                          rs_ssem.at[s], rs_rsem.at[s], (my_x, right_y, my_z))
            rdma.start()
            sends.append(rdma)
            remote(rs_buf.at[s], rs_buf.at[s], rs_ssem.at[s], rs_rsem.at[s],
                   (my_x, my_y, my_z)).wait_recv()
            sl = pl.ds(c_recv * CH, CH)
            acc[sl] = acc[sl] + rs_buf[s]

        c_own = (my_y + 1) % Y
        sl = pl.ds(c_own * CH, CH)
        yv = acc[sl].astype(jnp.float32) + r_ref[sl]
        ms = jnp.mean(yv * yv, axis=-1, keepdims=True)
        o_chunk[...] = (yv * lax.rsqrt(ms + 1e-6) * gm_ref[...]).astype(jnp.bfloat16)
        loc = pltpu.make_async_copy(o_chunk, buf.at[p * Y + c_own], loc_sem)
        loc.start()
        loc.wait()

        def c_of(k):
            return (my_y + 1 - k) % Y

        pending = []
        conv_state = [0]

        def convert_store(q, c):
            s_i = conv_state[0] % 2
            if len(pending) == 2:
                pending.pop(0).wait()
            cp = pltpu.make_async_copy(
                conv.at[s_i], out_ref.at[pl.ds(q * R + c * CH, CH)],
                conv_sems.at[s_i],
            )
            conv[s_i] = buf[q * Y + c].astype(jnp.float32)
            cp.start()
            pending.append(cp)
            conv_state[0] += 1

        me = (my_x, my_y, my_z)
        for r in range(N_CW + Y):
            arrivals = []
            if r == 0:
                src = buf.at[p * Y + c_own]
                ag = remote(src, src, ag_ssem.at[0], ag_rsem.at[0],
                            (my_x, right_y, my_z))
                ag.start()
                sends.append(ag)
                arrivals.append((p, c_own))
            if 1 <= r <= Y - 1:
                c_r = c_of(r)
                dst = buf.at[p * Y + c_r]
                remote(dst, dst, ag_ssem.at[r - 1], ag_rsem.at[r - 1],
                       me).wait_recv()
                if r <= Y - 2:
                    ag = remote(dst, dst, ag_ssem.at[r], ag_rsem.at[r],
                                (my_x, right_y, my_z))
                    ag.start()
                    sends.append(ag)
                arrivals.append((p, c_r))
            if r <= Y - 1:
                src = buf.at[p * Y + c_of(r)]
                cw = remote(src, src, cw_ssem.at[r, 0], cw_rsem.at[r, 0],
                            (rx, my_y, rz))
                cw.start()
                sends.append(cw)
                ccw = remote(src, src, ccw_ssem.at[r, 0], ccw_rsem.at[r, 0],
                             (lx, my_y, lz))
                ccw.start()
                sends.append(ccw)

            for k in range(Y):
                h = r - k
                if 1 <= h <= N_CW:
                    q = (p - h) % NG
                    ref = buf.at[q * Y + c_of(k)]
                    remote(ref, ref, cw_ssem.at[k, h - 1],
                           cw_rsem.at[k, h - 1], me).wait_recv()
                    arrivals.append((q, c_of(k)))
                    if h <= N_CW - 1:
                        cw = remote(ref, ref, cw_ssem.at[k, h],
                                    cw_rsem.at[k, h], (rx, my_y, rz))
                        cw.start()
                        sends.append(cw)
                if 1 <= h <= N_CCW:
                    q = (p + h) % NG
                    ref = buf.at[q * Y + c_of(k)]
                    remote(ref, ref, ccw_ssem.at[k, h - 1],
                           ccw_rsem.at[k, h - 1], me).wait_recv()
                    arrivals.append((q, c_of(k)))
                    if h <= N_CCW - 1:
                        ccw = remote(ref, ref, ccw_ssem.at[k, h],
                                     ccw_rsem.at[k, h], (lx, my_y, lz))
                        ccw.start()
                        sends.append(ccw)

            for q, c in arrivals:
                convert_store(q, c)

        for cp in pending:
            cp.wait()
        for rdma in sends:
            rdma.wait_send()

        @functools.partial(
            pl.run_scoped, second_barrier=pltpu.SemaphoreType.REGULAR
        )
        def _(second_barrier):
            for dev in neighbors:
                pl.semaphore_signal(
                    second_barrier, inc=1,
                    device_id=dev, device_id_type=pl.DeviceIdType.MESH,
                )
            pl.semaphore_wait(second_barrier, len(neighbors))

    return pl.pallas_call(
        body,
        out_shape=jax.ShapeDtypeStruct((M, D), jnp.float32),
        in_specs=[
            pl.BlockSpec(memory_space=pltpu.MemorySpace.VMEM),
            pl.BlockSpec(memory_space=pltpu.MemorySpace.VMEM),
            pl.BlockSpec(memory_space=pltpu.MemorySpace.VMEM),
        ],
        out_specs=pl.BlockSpec(memory_space=pl.ANY),
        scratch_shapes=[
            pltpu.VMEM((R, D), jnp.bfloat16),
            pltpu.VMEM((Y - 1, CH, D), jnp.bfloat16),
            pltpu.VMEM((CH, D), jnp.bfloat16),
            pltpu.VMEM((NG * Y, CH, D), jnp.bfloat16),
            pltpu.VMEM((2, CH, D), jnp.float32),
            pltpu.SemaphoreType.DMA((Y - 1,)),
            pltpu.SemaphoreType.DMA((Y - 1,)),
            pltpu.SemaphoreType.DMA((Y - 1,)),
            pltpu.SemaphoreType.DMA((Y - 1,)),
            pltpu.SemaphoreType.DMA((Y, N_CW)),
            pltpu.SemaphoreType.DMA((Y, N_CW)),
            pltpu.SemaphoreType.DMA((Y, N_CCW)),
            pltpu.SemaphoreType.DMA((Y, N_CCW)),
            pltpu.SemaphoreType.DMA,
            pltpu.SemaphoreType.DMA((2,)),
        ],
        compiler_params=pltpu.CompilerParams(
            collective_id=0, vmem_limit_bytes=100 * 1024 * 1024
        ),
    )(x_rows, resid_rows, gamma2d)


def kernel(partial, resid, gamma):
    my_x = lax.axis_index("x")
    my_z = lax.axis_index("z")
    g = jnp.where(my_x == 0, my_z, 7 - my_z)
    row0 = g * R
    x_rows = lax.dynamic_slice_in_dim(
        partial.reshape(M, D), row0, R, axis=0
    ).astype(jnp.bfloat16)
    resid_rows = lax.dynamic_slice_in_dim(resid, row0, R, axis=0)
    return _fused(x_rows, resid_rows, gamma.reshape(1, D))


# device time: 284727 ns/iter; 10.5494x vs baseline; 1.0416x over previous
import functools

import jax
import jax.numpy as jnp
from jax import lax
from jax.experimental import pallas as pl
from jax.experimental.pallas import tpu as pltpu

Y = 4
M = 4096
D = 4096
NG = 8
R = M // NG
CH = R // Y
N_CW = NG // 2
N_CCW = NG - 1 - N_CW


def _pos_to_xz(p):
    x = p // 4
    z = jnp.where(x == 0, p, 7 - p)
    return x, z


def _fused(x_rows, resid_rows, gamma2d):
    def body(part_ref, resid_ref, gm_ref, out_ref, acc, rs_buf, o_chunk, buf,
             conv, x_stage, r_stage, rs_ssem, rs_rsem, ag_ssem, ag_rsem,
             cw_ssem, cw_rsem, ccw_ssem, ccw_rsem, loc_sem, conv_sems,
             in_sems):
        my_x = lax.axis_index("x")
        my_y = lax.axis_index("y")
        my_z = lax.axis_index("z")
        p = jnp.where(my_x == 0, my_z, 7 - my_z)
        rx, rz = _pos_to_xz((p + 1) % NG)
        lx, lz = _pos_to_xz((p - 1) % NG)
        right_y = (my_y + 1) % Y
        left_y = (my_y - 1) % Y
        c_own = (my_y + 1) % Y

        xcp = pltpu.make_async_copy(
            part_ref.at[0, pl.ds(p * R, R)], x_stage, in_sems.at[0]
        )
        xcp.start()
        rcp = pltpu.make_async_copy(
            resid_ref.at[pl.ds(p * R + c_own * CH, CH)], r_stage, in_sems.at[1]
        )
        rcp.start()

        neighbors = (
            (my_x, left_y, my_z),
            (my_x, right_y, my_z),
            (rx, my_y, rz),
            (lx, my_y, lz),
        )
        barrier_sem = pltpu.get_barrier_semaphore()
        for dev in neighbors:
            pl.semaphore_signal(
                barrier_sem, inc=1,
                device_id=dev, device_id_type=pl.DeviceIdType.MESH,
            )
        pl.semaphore_wait(barrier_sem, len(neighbors))

        def remote(src, dst, ssem, rsem, dev):
            return pltpu.make_async_remote_copy(
                src_ref=src, dst_ref=dst, send_sem=ssem, recv_sem=rsem,
                device_id=dev, device_id_type=pl.DeviceIdType.MESH,
            )

        sends = []

        xcp.wait()
        acc[...] = x_stage[...].astype(jnp.bfloat16)
        for s in range(Y - 1):
            c_send = (my_y - s) % Y
            c_recv = (my_y - s - 1) % Y
            rdma = remote(acc.at[pl.ds(c_send * CH, CH)], rs_buf.at[s],
                          rs_ssem.at[s], rs_rsem.at[s], (my_x, right_y, my_z))
            rdma.start()
            sends.append(rdma)
            remote(rs_buf.at[s], rs_buf.at[s], rs_ssem.at[s], rs_rsem.at[s],
                   (my_x, my_y, my_z)).wait_recv()
            sl = pl.ds(c_recv * CH, CH)
            acc[sl] = acc[sl] + rs_buf[s]

        rcp.wait()
        sl = pl.ds(c_own * CH, CH)
        yv = acc[sl].astype(jnp.float32) + r_stage[...]
        ms = jnp.mean(yv * yv, axis=-1, keepdims=True)
        o_chunk[...] = (yv * lax.rsqrt(ms + 1e-6) * gm_ref[...]).astype(jnp.bfloat16)
        loc = pltpu.make_async_copy(o_chunk, buf.at[p * Y + c_own], loc_sem)
        loc.start()
        loc.wait()

        def c_of(k):
            return (my_y + 1 - k) % Y

        pending = []
        conv_state = [0]

        HR = R // 2

        def convert_store_group(q):
            for half in range(2):
                s_i = conv_state[0] % 2
                if len(pending) == 2:
                    pending.pop(0).wait()
                cp = pltpu.make_async_copy(
                    conv.at[s_i],
                    out_ref.at[pl.ds(q * R + half * HR, HR)],
                    conv_sems.at[s_i],
                )
                conv[s_i] = (
                    buf[pl.ds(q * Y + half * 2, 2)]
                    .reshape(HR, D)
                    .astype(jnp.float32)
                )
                cp.start()
                pending.append(cp)
                conv_state[0] += 1

        me = (my_x, my_y, my_z)
        for r in range(N_CW + Y):
            if r == 0:
                src = buf.at[p * Y + c_own]
                ag = remote(src, src, ag_ssem.at[0], ag_rsem.at[0],
                            (my_x, right_y, my_z))
                ag.start()
                sends.append(ag)
            if 1 <= r <= Y - 1:
                c_r = c_of(r)
                dst = buf.at[p * Y + c_r]
                remote(dst, dst, ag_ssem.at[r - 1], ag_rsem.at[r - 1],
                       me).wait_recv()
                if r <= Y - 2:
                    ag = remote(dst, dst, ag_ssem.at[r], ag_rsem.at[r],
                                (my_x, right_y, my_z))
                    ag.start()
                    sends.append(ag)
            if r <= Y - 1:
                src = buf.at[p * Y + c_of(r)]
                cw = remote(src, src, cw_ssem.at[r, 0], cw_rsem.at[r, 0],
                            (rx, my_y, rz))
                cw.start()
                sends.append(cw)
                ccw = remote(src, src, ccw_ssem.at[r, 0], ccw_rsem.at[r, 0],
                             (lx, my_y, lz))
                ccw.start()
                sends.append(ccw)

            for k in range(Y):
                h = r - k
                if 1 <= h <= N_CW:
                    q = (p - h) % NG
                    ref = buf.at[q * Y + c_of(k)]
                    remote(ref, ref, cw_ssem.at[k, h - 1],
                           cw_rsem.at[k, h - 1], me).wait_recv()
                    if h <= N_CW - 1:
                        cw = remote(ref, ref, cw_ssem.at[k, h],
                                    cw_rsem.at[k, h], (rx, my_y, rz))
                        cw.start()
                        sends.append(cw)
                if 1 <= h <= N_CCW:
                    q = (p + h) % NG
                    ref = buf.at[q * Y + c_of(k)]
                    remote(ref, ref, ccw_ssem.at[k, h - 1],
                           ccw_rsem.at[k, h - 1], me).wait_recv()
                    if h <= N_CCW - 1:
                        ccw = remote(ref, ref, ccw_ssem.at[k, h],
                                     ccw_rsem.at[k, h], (lx, my_y, lz))
                        ccw.start()
                        sends.append(ccw)

            if r == Y - 1:
                convert_store_group(p)
            d = r - (Y - 1)
            if 1 <= d <= N_CW:
                convert_store_group((p - d) % NG)
            if 1 <= d <= N_CCW:
                convert_store_group((p + d) % NG)

        for cp in pending:
            cp.wait()
        for rdma in sends:
            rdma.wait_send()

        @functools.partial(
            pl.run_scoped, second_barrier=pltpu.SemaphoreType.REGULAR
        )
        def _(second_barrier):
            for dev in neighbors:
                pl.semaphore_signal(
                    second_barrier, inc=1,
                    device_id=dev, device_id_type=pl.DeviceIdType.MESH,
                )
            pl.semaphore_wait(second_barrier, len(neighbors))

    return pl.pallas_call(
        body,
        out_shape=jax.ShapeDtypeStruct((M, D), jnp.float32),
        in_specs=[
            pl.BlockSpec(memory_space=pl.ANY),
            pl.BlockSpec(memory_space=pl.ANY),
            pl.BlockSpec(memory_space=pltpu.MemorySpace.VMEM),
        ],
        out_specs=pl.BlockSpec(memory_space=pl.ANY),
        scratch_shapes=[
            pltpu.VMEM((R, D), jnp.bfloat16),
            pltpu.VMEM((Y - 1, CH, D), jnp.bfloat16),
            pltpu.VMEM((CH, D), jnp.bfloat16),
            pltpu.VMEM((NG * Y, CH, D), jnp.bfloat16),
            pltpu.VMEM((2, R // 2, D), jnp.float32),
            pltpu.VMEM((R, D), jnp.float32),
            pltpu.VMEM((CH, D), jnp.float32),
            pltpu.SemaphoreType.DMA((Y - 1,)),
            pltpu.SemaphoreType.DMA((Y - 1,)),
            pltpu.SemaphoreType.DMA((Y - 1,)),
            pltpu.SemaphoreType.DMA((Y - 1,)),
            pltpu.SemaphoreType.DMA((Y, N_CW)),
            pltpu.SemaphoreType.DMA((Y, N_CW)),
            pltpu.SemaphoreType.DMA((Y, N_CCW)),
            pltpu.SemaphoreType.DMA((Y, N_CCW)),
            pltpu.SemaphoreType.DMA,
            pltpu.SemaphoreType.DMA((2,)),
            pltpu.SemaphoreType.DMA((2,)),
        ],
        compiler_params=pltpu.CompilerParams(
            collective_id=0, vmem_limit_bytes=100 * 1024 * 1024
        ),
    )(x_rows, resid_rows, gamma2d)


def kernel(partial, resid, gamma):
    return _fused(partial, resid, gamma.reshape(1, D))


# device time: 267429 ns/iter; 11.2317x vs baseline; 1.0647x over previous
import functools

import jax
import jax.numpy as jnp
from jax import lax
from jax.experimental import pallas as pl
from jax.experimental.pallas import tpu as pltpu

Y = 4
M = 4096
D = 4096
NG = 8
R = M // NG
CH = R // Y
N_CW = NG // 2
N_CCW = NG - 1 - N_CW


def _pos_to_xz(p):
    x = p // 4
    z = jnp.where(x == 0, p, 7 - p)
    return x, z


def _fused(x_rows, resid_rows, gamma2d):
    def body(part_ref, resid_ref, gm_ref, out_ref, acc, rs_buf, o_chunk, buf,
             conv, x_stage, r_stage, rs_ssem, rs_rsem, ag_ssem, ag_rsem,
             lng_ssem, lng_rsem, sht_ssem, sht_rsem, loc_sem, conv_sems,
             in_sems):
        my_x = lax.axis_index("x")
        my_y = lax.axis_index("y")
        my_z = lax.axis_index("z")
        p = jnp.where(my_x == 0, my_z, 7 - my_z)
        rx, rz = _pos_to_xz((p + 1) % NG)
        lx, lz = _pos_to_xz((p - 1) % NG)
        right_y = (my_y + 1) % Y
        left_y = (my_y - 1) % Y
        c_own = (my_y + 1) % Y

        xcp = pltpu.make_async_copy(
            part_ref.at[0, pl.ds(p * R, R)], x_stage, in_sems.at[0]
        )
        xcp.start()
        rcp = pltpu.make_async_copy(
            resid_ref.at[pl.ds(p * R + c_own * CH, CH)], r_stage, in_sems.at[1]
        )
        rcp.start()

        neighbors = (
            (my_x, left_y, my_z),
            (my_x, right_y, my_z),
            (rx, my_y, rz),
            (lx, my_y, lz),
        )
        barrier_sem = pltpu.get_barrier_semaphore()
        for dev in neighbors:
            pl.semaphore_signal(
                barrier_sem, inc=1,
                device_id=dev, device_id_type=pl.DeviceIdType.MESH,
            )
        pl.semaphore_wait(barrier_sem, len(neighbors))

        def remote(src, dst, ssem, rsem, dev):
            return pltpu.make_async_remote_copy(
                src_ref=src, dst_ref=dst, send_sem=ssem, recv_sem=rsem,
                device_id=dev, device_id_type=pl.DeviceIdType.MESH,
            )

        sends = []

        xcp.wait()
        acc[...] = x_stage[...].astype(jnp.bfloat16)
        for s in range(Y - 1):
            c_send = (my_y - s) % Y
            c_recv = (my_y - s - 1) % Y
            rdma = remote(acc.at[pl.ds(c_send * CH, CH)], rs_buf.at[s],
                          rs_ssem.at[s], rs_rsem.at[s], (my_x, right_y, my_z))
            rdma.start()
            sends.append(rdma)
            remote(rs_buf.at[s], rs_buf.at[s], rs_ssem.at[s], rs_rsem.at[s],
                   (my_x, my_y, my_z)).wait_recv()
            sl = pl.ds(c_recv * CH, CH)
            acc[sl] = acc[sl] + rs_buf[s]

        rcp.wait()
        sl = pl.ds(c_own * CH, CH)
        yv = acc[sl].astype(jnp.float32) + r_stage[...]
        ms = jnp.mean(yv * yv, axis=-1, keepdims=True)
        o_chunk[...] = (yv * lax.rsqrt(ms + 1e-6) * gm_ref[...]).astype(jnp.bfloat16)
        loc = pltpu.make_async_copy(o_chunk, buf.at[p * Y + c_own], loc_sem)
        loc.start()
        loc.wait()

        def c_of(k):
            return (my_y + 1 - k) % Y

        pending = []
        conv_state = [0]

        HR = R // 2

        def convert_store_group(q):
            for half in range(2):
                s_i = conv_state[0] % 2
                if len(pending) == 2:
                    pending.pop(0).wait()
                cp = pltpu.make_async_copy(
                    conv.at[s_i],
                    out_ref.at[pl.ds(q * R + half * HR, HR)],
                    conv_sems.at[s_i],
                )
                conv[s_i] = (
                    buf[pl.ds(q * Y + half * 2, 2)]
                    .reshape(HR, D)
                    .astype(jnp.float32)
                )
                cp.start()
                pending.append(cp)
                conv_state[0] += 1

        me = (my_x, my_y, my_z)
        for r in range(N_CW + Y):
            if r == 0:
                src = buf.at[p * Y + c_own]
                ag = remote(src, src, ag_ssem.at[0], ag_rsem.at[0],
                            (my_x, right_y, my_z))
                ag.start()
                sends.append(ag)
            if 1 <= r <= Y - 1:
                c_r = c_of(r)
                dst = buf.at[p * Y + c_r]
                remote(dst, dst, ag_ssem.at[r - 1], ag_rsem.at[r - 1],
                       me).wait_recv()
                if r <= Y - 2:
                    ag = remote(dst, dst, ag_ssem.at[r], ag_rsem.at[r],
                                (my_x, right_y, my_z))
                    ag.start()
                    sends.append(ag)
            def long_dev(k):
                return (rx, my_y, rz) if k % 2 == 0 else (lx, my_y, lz)

            def short_dev(k):
                return (lx, my_y, lz) if k % 2 == 0 else (rx, my_y, rz)

            def long_origin(k, h):
                return (p - h) % NG if k % 2 == 0 else (p + h) % NG

            def short_origin(k, h):
                return (p + h) % NG if k % 2 == 0 else (p - h) % NG

            if r <= Y - 1:
                src = buf.at[p * Y + c_of(r)]
                lng = remote(src, src, lng_ssem.at[r, 0], lng_rsem.at[r, 0],
                             long_dev(r))
                lng.start()
                sends.append(lng)
                sht = remote(src, src, sht_ssem.at[r, 0], sht_rsem.at[r, 0],
                             short_dev(r))
                sht.start()
                sends.append(sht)

            for k in range(Y):
                h = r - k
                if 1 <= h <= N_CW:
                    ref = buf.at[long_origin(k, h) * Y + c_of(k)]
                    remote(ref, ref, lng_ssem.at[k, h - 1],
                           lng_rsem.at[k, h - 1], me).wait_recv()
                    if h <= N_CW - 1:
                        lng = remote(ref, ref, lng_ssem.at[k, h],
                                     lng_rsem.at[k, h], long_dev(k))
                        lng.start()
                        sends.append(lng)
                if 1 <= h <= N_CCW:
                    ref = buf.at[short_origin(k, h) * Y + c_of(k)]
                    remote(ref, ref, sht_ssem.at[k, h - 1],
                           sht_rsem.at[k, h - 1], me).wait_recv()
                    if h <= N_CCW - 1:
                        sht = remote(ref, ref, sht_ssem.at[k, h],
                                     sht_rsem.at[k, h], short_dev(k))
                        sht.start()
                        sends.append(sht)

            if r == Y - 1:
                convert_store_group(p)
            d = r - (Y - 1)
            if 1 <= d <= N_CW:
                convert_store_group((p - d) % NG)
            if 1 <= d <= N_CCW:
                convert_store_group((p + d) % NG)

        for cp in pending:
            cp.wait()
        for rdma in sends:
            rdma.wait_send()

        @functools.partial(
            pl.run_scoped, second_barrier=pltpu.SemaphoreType.REGULAR
        )
        def _(second_barrier):
            for dev in neighbors:
                pl.semaphore_signal(
                    second_barrier, inc=1,
                    device_id=dev, device_id_type=pl.DeviceIdType.MESH,
                )
            pl.semaphore_wait(second_barrier, len(neighbors))

    return pl.pallas_call(
        body,
        out_shape=jax.ShapeDtypeStruct((M, D), jnp.float32),
        in_specs=[
            pl.BlockSpec(memory_space=pl.ANY),
            pl.BlockSpec(memory_space=pl.ANY),
            pl.BlockSpec(memory_space=pltpu.MemorySpace.VMEM),
        ],
        out_specs=pl.BlockSpec(memory_space=pl.ANY),
        scratch_shapes=[
            pltpu.VMEM((R, D), jnp.bfloat16),
            pltpu.VMEM((Y - 1, CH, D), jnp.bfloat16),
            pltpu.VMEM((CH, D), jnp.bfloat16),
            pltpu.VMEM((NG * Y, CH, D), jnp.bfloat16),
            pltpu.VMEM((2, R // 2, D), jnp.float32),
            pltpu.VMEM((R, D), jnp.float32),
            pltpu.VMEM((CH, D), jnp.float32),
            pltpu.SemaphoreType.DMA((Y - 1,)),
            pltpu.SemaphoreType.DMA((Y - 1,)),
            pltpu.SemaphoreType.DMA((Y - 1,)),
            pltpu.SemaphoreType.DMA((Y - 1,)),
            pltpu.SemaphoreType.DMA((Y, N_CW)),
            pltpu.SemaphoreType.DMA((Y, N_CW)),
            pltpu.SemaphoreType.DMA((Y, N_CCW)),
            pltpu.SemaphoreType.DMA((Y, N_CCW)),
            pltpu.SemaphoreType.DMA,
            pltpu.SemaphoreType.DMA((2,)),
            pltpu.SemaphoreType.DMA((2,)),
        ],
        compiler_params=pltpu.CompilerParams(
            collective_id=0, vmem_limit_bytes=100 * 1024 * 1024
        ),
    )(x_rows, resid_rows, gamma2d)


def kernel(partial, resid, gamma):
    return _fused(partial, resid, gamma.reshape(1, D))
